# Initial kernel scaffold; baseline (speedup 1.0000x reference)
import jax
import jax.numpy as jnp
from jax.experimental import pallas as pl


def kernel(x, edge_index, W, b):
    raise NotImplementedError("write your pallas kernel here")


# speedup vs baseline: 7.4007x; 7.4007x over previous
import functools

import jax
import jax.numpy as jnp
from jax import lax
from jax.experimental import pallas as pl
from jax.experimental.pallas import tpu as pltpu
from jax.experimental.pallas import tpu_sc as plsc

N = 10000
E = 320000
D_IN = 128
H = 64
K = 8

NC = 2
NS = 16
CH = 128
NCH = 80
EP = NC * NS * NCH * CH
TRASH = N
ACC = 10112
ZROWS = ACC // NS
WROWS = 624

@functools.cache
def _mesh():
    return plsc.VectorSubcoreMesh(core_axis_name="c", subcore_axis_name="s",
                                  num_cores=NC, num_subcores=NS)


_SC_PARAMS = pltpu.CompilerParams(use_tc_tiling_on_sc=False)


def _sc_prop(g, src4, dst4, zeros_h):
    return pl.kernel(
        _sc_prop_body,
        mesh=_mesh(),
        out_type=jax.ShapeDtypeStruct((NC, N, H), jnp.float32),
        scratch_types=[
            pltpu.VMEM((NCH, CH), jnp.int32),
            pltpu.VMEM((NCH, CH), jnp.int32),
            pltpu.VMEM((CH, H), jnp.float32),
            pltpu.VMEM((CH, H), jnp.float32),
            pltpu.VMEM_SHARED((ACC, H), jnp.float32),
            pltpu.SemaphoreType.DMA,
            pltpu.SemaphoreType.DMA,
        ],
        compiler_params=_SC_PARAMS,
    )(g, src4, dst4, zeros_h)


def _sc_prop_body(g_hbm, src_hbm, dst_hbm, zeros_hbm, s_out,
                  isrc, idst, gb0, gb1, acc, sem_a, sem_b):
    c = lax.axis_index("c")
    s = lax.axis_index("s")
    pltpu.sync_copy(zeros_hbm.at[pl.ds(s * ZROWS, ZROWS)],
                    acc.at[pl.ds(s * ZROWS, ZROWS)])
    pltpu.sync_copy(src_hbm.at[c, s], isrc)
    pltpu.sync_copy(dst_hbm.at[c, s], idst)
    plsc.subcore_barrier()

    @pl.loop(0, NCH // 2)
    def _(jj):
        j0 = jj * 2
        cp0 = pltpu.async_copy(g_hbm.at[isrc.at[j0]], gb0, sem_a)
        cp1 = pltpu.async_copy(g_hbm.at[isrc.at[j0 + 1]], gb1, sem_b)
        cp0.wait()
        pltpu.sync_copy(gb0, acc.at[idst.at[j0]], add=True)
        cp1.wait()
        pltpu.sync_copy(gb1, acc.at[idst.at[j0 + 1]], add=True)

    plsc.subcore_barrier()
    pltpu.sync_copy(acc.at[pl.ds(s * WROWS, WROWS)],
                    s_out.at[c, pl.ds(s * WROWS, WROWS)])

    @pl.when(s == 0)
    def _():
        pltpu.sync_copy(acc.at[pl.ds(NS * WROWS, N - NS * WROWS)],
                        s_out.at[c, pl.ds(NS * WROWS, N - NS * WROWS)])


def _sc_deg(src4, ones_16, zeros_16):
    return pl.kernel(
        _sc_deg_body,
        mesh=_mesh(),
        out_type=jax.ShapeDtypeStruct((NC, N, 16), jnp.float32),
        scratch_types=[
            pltpu.VMEM((NCH, CH), jnp.int32),
            pltpu.VMEM((CH, 16), jnp.float32),
            pltpu.VMEM_SHARED((ACC, 16), jnp.float32),
        ],
        compiler_params=_SC_PARAMS,
    )(src4, ones_16, zeros_16)


def _sc_deg_body(src_hbm, ones_hbm, zeros_hbm, d_out, isrc, ones_v, acc):
    c = lax.axis_index("c")
    s = lax.axis_index("s")
    pltpu.sync_copy(zeros_hbm.at[pl.ds(s * ZROWS, ZROWS)],
                    acc.at[pl.ds(s * ZROWS, ZROWS)])
    pltpu.sync_copy(src_hbm.at[c, s], isrc)
    pltpu.sync_copy(ones_hbm, ones_v)
    plsc.subcore_barrier()

    @pl.loop(0, NCH)
    def _(j):
        pltpu.sync_copy(ones_v, acc.at[isrc.at[j]], add=True)

    plsc.subcore_barrier()
    pltpu.sync_copy(acc.at[pl.ds(s * WROWS, WROWS)],
                    d_out.at[c, pl.ds(s * WROWS, WROWS)])

    @pl.when(s == 0)
    def _():
        pltpu.sync_copy(acc.at[pl.ds(NS * WROWS, N - NS * WROWS)],
                        d_out.at[c, pl.ds(NS * WROWS, N - NS * WROWS)])


BM = 2000
BD = 2000


def _mm_body(x_ref, w_ref, o_ref):
    o_ref[0] = jnp.dot(x_ref[...], w_ref[0],
                       preferred_element_type=jnp.float32)


def _matmul(x, W):
    return pl.pallas_call(
        _mm_body,
        grid=(N // BM, K),
        in_specs=[
            pl.BlockSpec((BM, D_IN), lambda i, k: (i, 0)),
            pl.BlockSpec((1, D_IN, H), lambda i, k: (k, 0, 0)),
        ],
        out_specs=pl.BlockSpec((1, BM, H), lambda i, k: (k, i, 0)),
        out_shape=jax.ShapeDtypeStruct((K, N, H), jnp.float32),
    )(x, W)


def _pre_body(deg_ref, a7_ref, dinv_ref, g_ref):
    deg = deg_ref[0, :, 0:1] + deg_ref[1, :, 0:1]
    dinv = jnp.where(deg > 0, lax.rsqrt(jnp.maximum(deg, 1.0)), 0.0)
    dinv_ref[...] = dinv
    g_ref[...] = dinv * a7_ref[0]


def _pre(deg_parts, a):
    return pl.pallas_call(
        _pre_body,
        grid=(N // BD,),
        in_specs=[
            pl.BlockSpec((NC, BD, 16), lambda i: (0, i, 0)),
            pl.BlockSpec((1, BD, H), lambda i: (K - 1, i, 0)),
        ],
        out_specs=[
            pl.BlockSpec((BD, 1), lambda i: (i, 0)),
            pl.BlockSpec((BD, H), lambda i: (i, 0)),
        ],
        out_shape=[
            jax.ShapeDtypeStruct((N, 1), jnp.float32),
            jax.ShapeDtypeStruct((N, H), jnp.float32),
        ],
    )(deg_parts, a)


def _dense_body(a_ref, s_ref, dinv_ref, bk2_ref, bk_ref, g_ref):
    ssum = s_ref[0] + s_ref[1]
    dinv = dinv_ref[...]
    bk = a_ref[0] - 2.0 * dinv * ssum - bk2_ref[...]
    bk_ref[...] = bk
    g_ref[...] = dinv * bk


def _dense_body_nob(a_ref, s_ref, dinv_ref, bk_ref, g_ref):
    ssum = s_ref[0] + s_ref[1]
    dinv = dinv_ref[...]
    bk = a_ref[0] - 2.0 * dinv * ssum
    bk_ref[...] = bk
    g_ref[...] = dinv * bk


def _dense(k, a, s_parts, dinv, bk2):
    in_specs = [
        pl.BlockSpec((1, BD, H), lambda i, k=k: (k, i, 0)),
        pl.BlockSpec((NC, BD, H), lambda i: (0, i, 0)),
        pl.BlockSpec((BD, 1), lambda i: (i, 0)),
    ]
    args = [a, s_parts, dinv]
    if bk2 is None:
        body = _dense_body_nob
    else:
        body = _dense_body
        in_specs.append(pl.BlockSpec((BD, H), lambda i: (i, 0)))
        args.append(bk2)
    return pl.pallas_call(
        body,
        grid=(N // BD,),
        in_specs=in_specs,
        out_specs=[
            pl.BlockSpec((BD, H), lambda i: (i, 0)),
            pl.BlockSpec((BD, H), lambda i: (i, 0)),
        ],
        out_shape=[
            jax.ShapeDtypeStruct((N, H), jnp.float32),
            jax.ShapeDtypeStruct((N, H), jnp.float32),
        ],
    )(*args)


def _final_body(a_ref, s_ref, dinv_ref, b2_ref, bias_ref, o_ref):
    ssum = s_ref[0] + s_ref[1]
    o_ref[...] = jnp.maximum(
        a_ref[0] - dinv_ref[...] * ssum - b2_ref[...] + bias_ref[...], 0.0)


def _final(a, s_parts, dinv, b2, bias2d):
    return pl.pallas_call(
        _final_body,
        grid=(N // BD,),
        in_specs=[
            pl.BlockSpec((1, BD, H), lambda i: (0, i, 0)),
            pl.BlockSpec((NC, BD, H), lambda i: (0, i, 0)),
            pl.BlockSpec((BD, 1), lambda i: (i, 0)),
            pl.BlockSpec((BD, H), lambda i: (i, 0)),
            pl.BlockSpec((1, H), lambda i: (0, 0)),
        ],
        out_specs=pl.BlockSpec((BD, H), lambda i: (i, 0)),
        out_shape=jax.ShapeDtypeStruct((N, H), jnp.float32),
    )(a, s_parts, dinv, b2, bias2d)


def kernel(x, edge_index, W, b):
    src = edge_index[0].astype(jnp.int32)
    dst = edge_index[1].astype(jnp.int32)
    pad = EP - E
    shape4 = (NC, NS, NCH, CH)
    src_g = jnp.concatenate([src, jnp.zeros((pad,), jnp.int32)]).reshape(shape4)
    src_s = jnp.concatenate([src, jnp.full((pad,), TRASH, jnp.int32)]).reshape(shape4)
    dst_s = jnp.concatenate([dst, jnp.full((pad,), TRASH, jnp.int32)]).reshape(shape4)

    bias2d = b.reshape(1, H)
    zeros_h = jnp.zeros((ACC, H), jnp.float32)
    zeros_16 = jnp.zeros((ACC, 16), jnp.float32)
    ones_16 = jnp.ones((CH, 16), jnp.float32)

    deg_parts = _sc_deg(src_s, ones_16, zeros_16)
    a = _matmul(x, W)
    dinv, g = _pre(deg_parts, a)

    b_prev2 = None
    b_prev1 = a[K - 1]
    for k in range(K - 2, 0, -1):
        s_parts = _sc_prop(g, src_g, dst_s, zeros_h)
        bk, g = _dense(k, a, s_parts, dinv, b_prev2)
        b_prev2, b_prev1 = b_prev1, bk

    s_parts = _sc_prop(g, src_g, dst_s, zeros_h)
    return _final(a, s_parts, dinv, b_prev2, bias2d)


# speedup vs baseline: 14.7054x; 1.9870x over previous
import functools

import jax
import jax.numpy as jnp
from jax import lax
from jax.experimental import pallas as pl
from jax.experimental.pallas import tpu as pltpu
from jax.experimental.pallas import tpu_sc as plsc

N = 10000
E = 320000
D_IN = 128
H = 64
K = 8

NC = 2
NS = 16
CH = 128
NCH = 80
EP = NC * NS * NCH * CH
TRASH = N
ACC = 10112
ZROWS = ACC // NS
WROWS = 624

@functools.cache
def _mesh():
    return plsc.VectorSubcoreMesh(core_axis_name="c", subcore_axis_name="s",
                                  num_cores=NC, num_subcores=NS)


_SC_PARAMS = pltpu.CompilerParams(use_tc_tiling_on_sc=False)


def _sc_prop(g, src4, dst4, zeros_h):
    return pl.kernel(
        _sc_prop_body,
        mesh=_mesh(),
        out_type=jax.ShapeDtypeStruct((NC, N, H), jnp.float32),
        scratch_types=[
            pltpu.VMEM((NCH, CH), jnp.int32),
            pltpu.VMEM((NCH, CH), jnp.int32),
            pltpu.VMEM((CH, H), jnp.float32),
            pltpu.VMEM((CH, H), jnp.float32),
            pltpu.VMEM_SHARED((ACC, H), jnp.float32),
            pltpu.VMEM_SHARED((N, H), jnp.float32),
            pltpu.SemaphoreType.DMA,
            pltpu.SemaphoreType.DMA,
        ],
        compiler_params=_SC_PARAMS,
    )(g, src4, dst4, zeros_h)


def _sc_prop_body(g_hbm, src_hbm, dst_hbm, zeros_hbm, s_out,
                  isrc, idst, gb0, gb1, acc, gsh, sem_a, sem_b):
    c = lax.axis_index("c")
    s = lax.axis_index("s")
    pltpu.sync_copy(zeros_hbm.at[pl.ds(s * ZROWS, ZROWS)],
                    acc.at[pl.ds(s * ZROWS, ZROWS)])
    pltpu.sync_copy(g_hbm.at[pl.ds(s * WROWS, WROWS)],
                    gsh.at[pl.ds(s * WROWS, WROWS)])

    @pl.when(s == 0)
    def _():
        pltpu.sync_copy(g_hbm.at[pl.ds(NS * WROWS, N - NS * WROWS)],
                        gsh.at[pl.ds(NS * WROWS, N - NS * WROWS)])

    pltpu.sync_copy(src_hbm.at[c, s], isrc)
    pltpu.sync_copy(dst_hbm.at[c, s], idst)
    plsc.subcore_barrier()

    @pl.loop(0, NCH // 2)
    def _(jj):
        j0 = jj * 2
        cp0 = pltpu.async_copy(gsh.at[isrc.at[j0]], gb0, sem_a)
        cp1 = pltpu.async_copy(gsh.at[isrc.at[j0 + 1]], gb1, sem_b)
        cp0.wait()
        pltpu.sync_copy(gb0, acc.at[idst.at[j0]], add=True)
        cp1.wait()
        pltpu.sync_copy(gb1, acc.at[idst.at[j0 + 1]], add=True)

    plsc.subcore_barrier()
    pltpu.sync_copy(acc.at[pl.ds(s * WROWS, WROWS)],
                    s_out.at[c, pl.ds(s * WROWS, WROWS)])

    @pl.when(s == 0)
    def _():
        pltpu.sync_copy(acc.at[pl.ds(NS * WROWS, N - NS * WROWS)],
                        s_out.at[c, pl.ds(NS * WROWS, N - NS * WROWS)])


def _sc_deg(src4, ones_16, zeros_16):
    return pl.kernel(
        _sc_deg_body,
        mesh=_mesh(),
        out_type=jax.ShapeDtypeStruct((NC, N, 16), jnp.float32),
        scratch_types=[
            pltpu.VMEM((NCH, CH), jnp.int32),
            pltpu.VMEM((CH, 16), jnp.float32),
            pltpu.VMEM_SHARED((ACC, 16), jnp.float32),
        ],
        compiler_params=_SC_PARAMS,
    )(src4, ones_16, zeros_16)


def _sc_deg_body(src_hbm, ones_hbm, zeros_hbm, d_out, isrc, ones_v, acc):
    c = lax.axis_index("c")
    s = lax.axis_index("s")
    pltpu.sync_copy(zeros_hbm.at[pl.ds(s * ZROWS, ZROWS)],
                    acc.at[pl.ds(s * ZROWS, ZROWS)])
    pltpu.sync_copy(src_hbm.at[c, s], isrc)
    pltpu.sync_copy(ones_hbm, ones_v)
    plsc.subcore_barrier()

    @pl.loop(0, NCH)
    def _(j):
        pltpu.sync_copy(ones_v, acc.at[isrc.at[j]], add=True)

    plsc.subcore_barrier()
    pltpu.sync_copy(acc.at[pl.ds(s * WROWS, WROWS)],
                    d_out.at[c, pl.ds(s * WROWS, WROWS)])

    @pl.when(s == 0)
    def _():
        pltpu.sync_copy(acc.at[pl.ds(NS * WROWS, N - NS * WROWS)],
                        d_out.at[c, pl.ds(NS * WROWS, N - NS * WROWS)])


BM = 2000
BD = 2000


def _mm_body(x_ref, w_ref, o_ref):
    o_ref[0] = jnp.dot(x_ref[...], w_ref[0],
                       preferred_element_type=jnp.float32)


def _matmul(x, W):
    return pl.pallas_call(
        _mm_body,
        grid=(N // BM, K),
        in_specs=[
            pl.BlockSpec((BM, D_IN), lambda i, k: (i, 0)),
            pl.BlockSpec((1, D_IN, H), lambda i, k: (k, 0, 0)),
        ],
        out_specs=pl.BlockSpec((1, BM, H), lambda i, k: (k, i, 0)),
        out_shape=jax.ShapeDtypeStruct((K, N, H), jnp.float32),
    )(x, W)


def _pre_body(deg_ref, a7_ref, dinv_ref, g_ref):
    deg = deg_ref[0, :, 0:1] + deg_ref[1, :, 0:1]
    dinv = jnp.where(deg > 0, lax.rsqrt(jnp.maximum(deg, 1.0)), 0.0)
    dinv_ref[...] = dinv
    g_ref[...] = dinv * a7_ref[0]


def _pre(deg_parts, a):
    return pl.pallas_call(
        _pre_body,
        grid=(N // BD,),
        in_specs=[
            pl.BlockSpec((NC, BD, 16), lambda i: (0, i, 0)),
            pl.BlockSpec((1, BD, H), lambda i: (K - 1, i, 0)),
        ],
        out_specs=[
            pl.BlockSpec((BD, 1), lambda i: (i, 0)),
            pl.BlockSpec((BD, H), lambda i: (i, 0)),
        ],
        out_shape=[
            jax.ShapeDtypeStruct((N, 1), jnp.float32),
            jax.ShapeDtypeStruct((N, H), jnp.float32),
        ],
    )(deg_parts, a)


def _dense_body(a_ref, s_ref, dinv_ref, bk2_ref, bk_ref, g_ref):
    ssum = s_ref[0] + s_ref[1]
    dinv = dinv_ref[...]
    bk = a_ref[0] - 2.0 * dinv * ssum - bk2_ref[...]
    bk_ref[...] = bk
    g_ref[...] = dinv * bk


def _dense_body_nob(a_ref, s_ref, dinv_ref, bk_ref, g_ref):
    ssum = s_ref[0] + s_ref[1]
    dinv = dinv_ref[...]
    bk = a_ref[0] - 2.0 * dinv * ssum
    bk_ref[...] = bk
    g_ref[...] = dinv * bk


def _dense(k, a, s_parts, dinv, bk2):
    in_specs = [
        pl.BlockSpec((1, BD, H), lambda i, k=k: (k, i, 0)),
        pl.BlockSpec((NC, BD, H), lambda i: (0, i, 0)),
        pl.BlockSpec((BD, 1), lambda i: (i, 0)),
    ]
    args = [a, s_parts, dinv]
    if bk2 is None:
        body = _dense_body_nob
    else:
        body = _dense_body
        in_specs.append(pl.BlockSpec((BD, H), lambda i: (i, 0)))
        args.append(bk2)
    return pl.pallas_call(
        body,
        grid=(N // BD,),
        in_specs=in_specs,
        out_specs=[
            pl.BlockSpec((BD, H), lambda i: (i, 0)),
            pl.BlockSpec((BD, H), lambda i: (i, 0)),
        ],
        out_shape=[
            jax.ShapeDtypeStruct((N, H), jnp.float32),
            jax.ShapeDtypeStruct((N, H), jnp.float32),
        ],
    )(*args)


def _final_body(a_ref, s_ref, dinv_ref, b2_ref, bias_ref, o_ref):
    ssum = s_ref[0] + s_ref[1]
    o_ref[...] = jnp.maximum(
        a_ref[0] - dinv_ref[...] * ssum - b2_ref[...] + bias_ref[...], 0.0)


def _final(a, s_parts, dinv, b2, bias2d):
    return pl.pallas_call(
        _final_body,
        grid=(N // BD,),
        in_specs=[
            pl.BlockSpec((1, BD, H), lambda i: (0, i, 0)),
            pl.BlockSpec((NC, BD, H), lambda i: (0, i, 0)),
            pl.BlockSpec((BD, 1), lambda i: (i, 0)),
            pl.BlockSpec((BD, H), lambda i: (i, 0)),
            pl.BlockSpec((1, H), lambda i: (0, 0)),
        ],
        out_specs=pl.BlockSpec((BD, H), lambda i: (i, 0)),
        out_shape=jax.ShapeDtypeStruct((N, H), jnp.float32),
    )(a, s_parts, dinv, b2, bias2d)


def kernel(x, edge_index, W, b):
    src = edge_index[0].astype(jnp.int32)
    dst = edge_index[1].astype(jnp.int32)
    pad = EP - E
    shape4 = (NC, NS, NCH, CH)
    src_g = jnp.concatenate([src, jnp.zeros((pad,), jnp.int32)]).reshape(shape4)
    src_s = jnp.concatenate([src, jnp.full((pad,), TRASH, jnp.int32)]).reshape(shape4)
    dst_s = jnp.concatenate([dst, jnp.full((pad,), TRASH, jnp.int32)]).reshape(shape4)

    bias2d = b.reshape(1, H)
    zeros_h = jnp.zeros((ACC, H), jnp.float32)
    zeros_16 = jnp.zeros((ACC, 16), jnp.float32)
    ones_16 = jnp.ones((CH, 16), jnp.float32)

    deg_parts = _sc_deg(src_s, ones_16, zeros_16)
    a = _matmul(x, W)
    dinv, g = _pre(deg_parts, a)

    b_prev2 = None
    b_prev1 = a[K - 1]
    for k in range(K - 2, 0, -1):
        s_parts = _sc_prop(g, src_g, dst_s, zeros_h)
        bk, g = _dense(k, a, s_parts, dinv, b_prev2)
        b_prev2, b_prev1 = b_prev1, bk

    s_parts = _sc_prop(g, src_g, dst_s, zeros_h)
    return _final(a, s_parts, dinv, b_prev2, bias2d)


# speedup vs baseline: 17.3432x; 1.1794x over previous
import functools

import jax
import jax.numpy as jnp
from jax import lax
from jax.experimental import pallas as pl
from jax.experimental.pallas import tpu as pltpu
from jax.experimental.pallas import tpu_sc as plsc

N = 10000
E = 320000
D_IN = 128
H = 64
K = 8

NC = 2
NS = 16
CH = 128
NCH = 80
EP = NC * NS * NCH * CH
TRASH = N
ACC = 10112
ZROWS = ACC // NS
WROWS = 624

@functools.cache
def _mesh():
    return plsc.VectorSubcoreMesh(core_axis_name="c", subcore_axis_name="s",
                                  num_cores=NC, num_subcores=NS)


_SC_PARAMS = pltpu.CompilerParams(use_tc_tiling_on_sc=False)


def _sc_prop(g, src4, dst4, zeros_h):
    return pl.kernel(
        _sc_prop_body,
        mesh=_mesh(),
        out_type=jax.ShapeDtypeStruct((NC, N, H), jnp.float32),
        scratch_types=[
            pltpu.VMEM((NCH, CH), jnp.int32),
            pltpu.VMEM((NCH, CH), jnp.int32),
            pltpu.VMEM((CH, H), jnp.float32),
            pltpu.VMEM((CH, H), jnp.float32),
            pltpu.VMEM((CH, H), jnp.float32),
            pltpu.VMEM((CH, H), jnp.float32),
            pltpu.VMEM_SHARED((ACC, H), jnp.float32),
            pltpu.VMEM_SHARED((N, H), jnp.float32),
            pltpu.SemaphoreType.DMA,
            pltpu.SemaphoreType.DMA,
            pltpu.SemaphoreType.DMA,
            pltpu.SemaphoreType.DMA,
            pltpu.SemaphoreType.DMA,
            pltpu.SemaphoreType.DMA,
            pltpu.SemaphoreType.DMA,
            pltpu.SemaphoreType.DMA,
        ],
        compiler_params=_SC_PARAMS,
    )(g, src4, dst4, zeros_h)


def _sc_prop_body(g_hbm, src_hbm, dst_hbm, zeros_hbm, s_out,
                  isrc, idst, gb0, gb1, gb2, gb3, acc, gsh,
                  gs0, gs1, gs2, gs3, ss0, ss1, ss2, ss3):
    gbufs = (gb0, gb1, gb2, gb3)
    gsems = (gs0, gs1, gs2, gs3)
    ssems = (ss0, ss1, ss2, ss3)
    c = lax.axis_index("c")
    s = lax.axis_index("s")
    pltpu.sync_copy(zeros_hbm.at[pl.ds(s * ZROWS, ZROWS)],
                    acc.at[pl.ds(s * ZROWS, ZROWS)])
    pltpu.sync_copy(g_hbm.at[pl.ds(s * WROWS, WROWS)],
                    gsh.at[pl.ds(s * WROWS, WROWS)])

    @pl.when(s == 0)
    def _():
        pltpu.sync_copy(g_hbm.at[pl.ds(NS * WROWS, N - NS * WROWS)],
                        gsh.at[pl.ds(NS * WROWS, N - NS * WROWS)])

    pltpu.sync_copy(src_hbm.at[c, s], isrc)
    pltpu.sync_copy(dst_hbm.at[c, s], idst)
    plsc.subcore_barrier()

    @pl.loop(0, NCH // 2)
    def _(jj):
        j0 = jj * 2
        for i in range(2):
            @pl.when(jj > 0)
            def _(i=i):
                pltpu.make_async_copy(gbufs[i], acc.at[idst.at[j0 - 2 + i]],
                                      ssems[i]).wait()
            pltpu.async_copy(gsh.at[isrc.at[j0 + i]], gbufs[i], gsems[i])
        for i in range(2):
            pltpu.make_async_copy(gsh.at[isrc.at[j0 + i]], gbufs[i],
                                  gsems[i]).wait()
            pltpu.async_copy(gbufs[i], acc.at[idst.at[j0 + i]], ssems[i],
                             add=True)

    for i in range(2):
        pltpu.make_async_copy(gbufs[i], acc.at[idst.at[NCH - 2 + i]],
                              ssems[i]).wait()

    plsc.subcore_barrier()
    pltpu.sync_copy(acc.at[pl.ds(s * WROWS, WROWS)],
                    s_out.at[c, pl.ds(s * WROWS, WROWS)])

    @pl.when(s == 0)
    def _():
        pltpu.sync_copy(acc.at[pl.ds(NS * WROWS, N - NS * WROWS)],
                        s_out.at[c, pl.ds(NS * WROWS, N - NS * WROWS)])


def _sc_deg(src4, ones_16, zeros_16):
    return pl.kernel(
        _sc_deg_body,
        mesh=_mesh(),
        out_type=jax.ShapeDtypeStruct((NC, N, 16), jnp.float32),
        scratch_types=[
            pltpu.VMEM((NCH, CH), jnp.int32),
            pltpu.VMEM((CH, 16), jnp.float32),
            pltpu.VMEM_SHARED((ACC, 16), jnp.float32),
            pltpu.SemaphoreType.DMA,
        ],
        compiler_params=_SC_PARAMS,
    )(src4, ones_16, zeros_16)


def _sc_deg_body(src_hbm, ones_hbm, zeros_hbm, d_out, isrc, ones_v, acc, sem):
    c = lax.axis_index("c")
    s = lax.axis_index("s")
    pltpu.sync_copy(zeros_hbm.at[pl.ds(s * ZROWS, ZROWS)],
                    acc.at[pl.ds(s * ZROWS, ZROWS)])
    pltpu.sync_copy(src_hbm.at[c, s], isrc)
    pltpu.sync_copy(ones_hbm, ones_v)
    plsc.subcore_barrier()

    @pl.loop(0, NCH)
    def _(j):
        pltpu.sync_copy(ones_v, acc.at[isrc.at[j]], add=True)

    plsc.subcore_barrier()
    pltpu.sync_copy(acc.at[pl.ds(s * WROWS, WROWS)],
                    d_out.at[c, pl.ds(s * WROWS, WROWS)])

    @pl.when(s == 0)
    def _():
        pltpu.sync_copy(acc.at[pl.ds(NS * WROWS, N - NS * WROWS)],
                        d_out.at[c, pl.ds(NS * WROWS, N - NS * WROWS)])


BM = 2000
BD = 2000


def _mm_body(x_ref, w_ref, o_ref):
    o_ref[0] = jnp.dot(x_ref[...], w_ref[0],
                       preferred_element_type=jnp.float32)


def _matmul(x, W):
    return pl.pallas_call(
        _mm_body,
        grid=(N // BM, K),
        in_specs=[
            pl.BlockSpec((BM, D_IN), lambda i, k: (i, 0)),
            pl.BlockSpec((1, D_IN, H), lambda i, k: (k, 0, 0)),
        ],
        out_specs=pl.BlockSpec((1, BM, H), lambda i, k: (k, i, 0)),
        out_shape=jax.ShapeDtypeStruct((K, N, H), jnp.float32),
    )(x, W)


def _pre_body(deg_ref, a7_ref, dinv_ref, g_ref):
    deg = deg_ref[0, :, 0:1] + deg_ref[1, :, 0:1]
    dinv = jnp.where(deg > 0, lax.rsqrt(jnp.maximum(deg, 1.0)), 0.0)
    dinv_ref[...] = dinv
    g_ref[...] = dinv * a7_ref[0]


def _pre(deg_parts, a):
    return pl.pallas_call(
        _pre_body,
        grid=(N // BD,),
        in_specs=[
            pl.BlockSpec((NC, BD, 16), lambda i: (0, i, 0)),
            pl.BlockSpec((1, BD, H), lambda i: (K - 1, i, 0)),
        ],
        out_specs=[
            pl.BlockSpec((BD, 1), lambda i: (i, 0)),
            pl.BlockSpec((BD, H), lambda i: (i, 0)),
        ],
        out_shape=[
            jax.ShapeDtypeStruct((N, 1), jnp.float32),
            jax.ShapeDtypeStruct((N, H), jnp.float32),
        ],
    )(deg_parts, a)


def _dense_body(a_ref, s_ref, dinv_ref, bk2_ref, bk_ref, g_ref):
    ssum = s_ref[0] + s_ref[1]
    dinv = dinv_ref[...]
    bk = a_ref[0] - 2.0 * dinv * ssum - bk2_ref[...]
    bk_ref[...] = bk
    g_ref[...] = dinv * bk


def _dense_body_nob(a_ref, s_ref, dinv_ref, bk_ref, g_ref):
    ssum = s_ref[0] + s_ref[1]
    dinv = dinv_ref[...]
    bk = a_ref[0] - 2.0 * dinv * ssum
    bk_ref[...] = bk
    g_ref[...] = dinv * bk


def _dense(k, a, s_parts, dinv, bk2):
    in_specs = [
        pl.BlockSpec((1, BD, H), lambda i, k=k: (k, i, 0)),
        pl.BlockSpec((NC, BD, H), lambda i: (0, i, 0)),
        pl.BlockSpec((BD, 1), lambda i: (i, 0)),
    ]
    args = [a, s_parts, dinv]
    if bk2 is None:
        body = _dense_body_nob
    else:
        body = _dense_body
        in_specs.append(pl.BlockSpec((BD, H), lambda i: (i, 0)))
        args.append(bk2)
    return pl.pallas_call(
        body,
        grid=(N // BD,),
        in_specs=in_specs,
        out_specs=[
            pl.BlockSpec((BD, H), lambda i: (i, 0)),
            pl.BlockSpec((BD, H), lambda i: (i, 0)),
        ],
        out_shape=[
            jax.ShapeDtypeStruct((N, H), jnp.float32),
            jax.ShapeDtypeStruct((N, H), jnp.float32),
        ],
    )(*args)


def _final_body(a_ref, s_ref, dinv_ref, b2_ref, bias_ref, o_ref):
    ssum = s_ref[0] + s_ref[1]
    o_ref[...] = jnp.maximum(
        a_ref[0] - dinv_ref[...] * ssum - b2_ref[...] + bias_ref[...], 0.0)


def _final(a, s_parts, dinv, b2, bias2d):
    return pl.pallas_call(
        _final_body,
        grid=(N // BD,),
        in_specs=[
            pl.BlockSpec((1, BD, H), lambda i: (0, i, 0)),
            pl.BlockSpec((NC, BD, H), lambda i: (0, i, 0)),
            pl.BlockSpec((BD, 1), lambda i: (i, 0)),
            pl.BlockSpec((BD, H), lambda i: (i, 0)),
            pl.BlockSpec((1, H), lambda i: (0, 0)),
        ],
        out_specs=pl.BlockSpec((BD, H), lambda i: (i, 0)),
        out_shape=jax.ShapeDtypeStruct((N, H), jnp.float32),
    )(a, s_parts, dinv, b2, bias2d)


def kernel(x, edge_index, W, b):
    src = edge_index[0].astype(jnp.int32)
    dst = edge_index[1].astype(jnp.int32)
    pad = EP - E
    shape4 = (NC, NS, NCH, CH)
    src_g = jnp.concatenate([src, jnp.zeros((pad,), jnp.int32)]).reshape(shape4)
    src_s = jnp.concatenate([src, jnp.full((pad,), TRASH, jnp.int32)]).reshape(shape4)
    dst_s = jnp.concatenate([dst, jnp.full((pad,), TRASH, jnp.int32)]).reshape(shape4)

    bias2d = b.reshape(1, H)
    zeros_h = jnp.zeros((ACC, H), jnp.float32)
    zeros_16 = jnp.zeros((ACC, 16), jnp.float32)
    ones_16 = jnp.ones((CH, 16), jnp.float32)

    deg_parts = _sc_deg(src_s, ones_16, zeros_16)
    a = _matmul(x, W)
    dinv, g = _pre(deg_parts, a)

    b_prev2 = None
    b_prev1 = a[K - 1]
    for k in range(K - 2, 0, -1):
        s_parts = _sc_prop(g, src_g, dst_s, zeros_h)
        bk, g = _dense(k, a, s_parts, dinv, b_prev2)
        b_prev2, b_prev1 = b_prev1, bk

    s_parts = _sc_prop(g, src_g, dst_s, zeros_h)
    return _final(a, s_parts, dinv, b_prev2, bias2d)


# speedup vs baseline: 18.3987x; 1.0609x over previous
import functools

import jax
import jax.numpy as jnp
from jax import lax
from jax.experimental import pallas as pl
from jax.experimental.pallas import tpu as pltpu
from jax.experimental.pallas import tpu_sc as plsc

N = 10000
E = 320000
D_IN = 128
H = 64
K = 8

NC = 2
NS = 16
CH = 128
NCH = 80
NCH2 = 158
HH = H // 2
EP2 = NS * NCH2 * CH
EP = NC * NS * NCH * CH
TRASH = N
ACC = 10112
ZROWS = ACC // NS
WROWS = 624

@functools.cache
def _mesh():
    return plsc.VectorSubcoreMesh(core_axis_name="c", subcore_axis_name="s",
                                  num_cores=NC, num_subcores=NS)


_SC_PARAMS = pltpu.CompilerParams(use_tc_tiling_on_sc=False)


def _sc_prop(g, src4, dst4, zeros_h):
    return pl.kernel(
        _sc_prop_body,
        mesh=_mesh(),
        out_type=jax.ShapeDtypeStruct((NC, N, HH), jnp.float32),
        scratch_types=[
            pltpu.VMEM((NCH2, CH), jnp.int32),
            pltpu.VMEM((NCH2, CH), jnp.int32),
            pltpu.VMEM((CH, HH), jnp.float32),
            pltpu.VMEM((CH, HH), jnp.float32),
            pltpu.VMEM((CH, HH), jnp.float32),
            pltpu.VMEM((CH, HH), jnp.float32),
            pltpu.VMEM_SHARED((ACC, HH), jnp.float32),
            pltpu.VMEM_SHARED((N, HH), jnp.float32),
            pltpu.SemaphoreType.DMA,
            pltpu.SemaphoreType.DMA,
            pltpu.SemaphoreType.DMA,
            pltpu.SemaphoreType.DMA,
            pltpu.SemaphoreType.DMA,
            pltpu.SemaphoreType.DMA,
            pltpu.SemaphoreType.DMA,
            pltpu.SemaphoreType.DMA,
        ],
        compiler_params=_SC_PARAMS,
    )(g, src4, dst4, zeros_h)


def _sc_prop_body(g_hbm, src_hbm, dst_hbm, zeros_hbm, s_out,
                  isrc, idst, gb0, gb1, gb2, gb3, acc, gsh,
                  gs0, gs1, gs2, gs3, ss0, ss1, ss2, ss3):
    gbufs = (gb0, gb1, gb2, gb3)
    gsems = (gs0, gs1, gs2, gs3)
    ssems = (ss0, ss1, ss2, ss3)
    c = lax.axis_index("c")
    s = lax.axis_index("s")
    cz = pltpu.async_copy(zeros_hbm.at[pl.ds(s * ZROWS, ZROWS)],
                          acc.at[pl.ds(s * ZROWS, ZROWS)], gs0)
    cg = pltpu.async_copy(g_hbm.at[c, pl.ds(s * WROWS, WROWS)],
                          gsh.at[pl.ds(s * WROWS, WROWS)], gs1)
    ci = pltpu.async_copy(src_hbm.at[s], isrc, gs2)
    cj = pltpu.async_copy(dst_hbm.at[s], idst, gs3)

    @pl.when(s == 0)
    def _():
        pltpu.sync_copy(g_hbm.at[c, pl.ds(NS * WROWS, N - NS * WROWS)],
                        gsh.at[pl.ds(NS * WROWS, N - NS * WROWS)])

    cz.wait()
    cg.wait()
    ci.wait()
    cj.wait()
    plsc.subcore_barrier()

    @pl.loop(0, NCH2 // 2)
    def _(jj):
        j0 = jj * 2
        for i in range(2):
            @pl.when(jj > 0)
            def _(i=i):
                pltpu.make_async_copy(gbufs[i], acc.at[idst.at[j0 - 2 + i]],
                                      ssems[i]).wait()
            pltpu.async_copy(gsh.at[isrc.at[j0 + i]], gbufs[i], gsems[i])
        for i in range(2):
            pltpu.make_async_copy(gsh.at[isrc.at[j0 + i]], gbufs[i],
                                  gsems[i]).wait()
            pltpu.async_copy(gbufs[i], acc.at[idst.at[j0 + i]], ssems[i],
                             add=True)

    for i in range(2):
        pltpu.make_async_copy(gbufs[i], acc.at[idst.at[NCH2 - 2 + i]],
                              ssems[i]).wait()

    plsc.subcore_barrier()
    pltpu.sync_copy(acc.at[pl.ds(s * WROWS, WROWS)],
                    s_out.at[c, pl.ds(s * WROWS, WROWS)])

    @pl.when(s == 0)
    def _():
        pltpu.sync_copy(acc.at[pl.ds(NS * WROWS, N - NS * WROWS)],
                        s_out.at[c, pl.ds(NS * WROWS, N - NS * WROWS)])


def _sc_deg(src4, ones_16, zeros_16):
    return pl.kernel(
        _sc_deg_body,
        mesh=_mesh(),
        out_type=jax.ShapeDtypeStruct((NC, N, 16), jnp.float32),
        scratch_types=[
            pltpu.VMEM((NCH, CH), jnp.int32),
            pltpu.VMEM((CH, 16), jnp.float32),
            pltpu.VMEM_SHARED((ACC, 16), jnp.float32),
            pltpu.SemaphoreType.DMA,
        ],
        compiler_params=_SC_PARAMS,
    )(src4, ones_16, zeros_16)


def _sc_deg_body(src_hbm, ones_hbm, zeros_hbm, d_out, isrc, ones_v, acc, sem):
    c = lax.axis_index("c")
    s = lax.axis_index("s")
    pltpu.sync_copy(zeros_hbm.at[pl.ds(s * ZROWS, ZROWS)],
                    acc.at[pl.ds(s * ZROWS, ZROWS)])
    pltpu.sync_copy(src_hbm.at[c, s], isrc)
    pltpu.sync_copy(ones_hbm, ones_v)
    plsc.subcore_barrier()

    @pl.loop(0, NCH)
    def _(j):
        pltpu.sync_copy(ones_v, acc.at[isrc.at[j]], add=True)

    plsc.subcore_barrier()
    pltpu.sync_copy(acc.at[pl.ds(s * WROWS, WROWS)],
                    d_out.at[c, pl.ds(s * WROWS, WROWS)])

    @pl.when(s == 0)
    def _():
        pltpu.sync_copy(acc.at[pl.ds(NS * WROWS, N - NS * WROWS)],
                        d_out.at[c, pl.ds(NS * WROWS, N - NS * WROWS)])


BM = 2000
BD = 2000


def _mm_body(x_ref, w_ref, o_ref):
    o_ref[0] = jnp.dot(x_ref[...], w_ref[0],
                       preferred_element_type=jnp.float32)


def _matmul(x, W):
    return pl.pallas_call(
        _mm_body,
        grid=(N // BM, K),
        in_specs=[
            pl.BlockSpec((BM, D_IN), lambda i, k: (i, 0)),
            pl.BlockSpec((1, D_IN, H), lambda i, k: (k, 0, 0)),
        ],
        out_specs=pl.BlockSpec((1, BM, H), lambda i, k: (k, i, 0)),
        out_shape=jax.ShapeDtypeStruct((K, N, H), jnp.float32),
    )(x, W)


def _pre_body(deg_ref, a7_ref, dinv_ref, g_ref):
    deg = deg_ref[0, :, 0:1] + deg_ref[1, :, 0:1]
    dinv = jnp.where(deg > 0, lax.rsqrt(jnp.maximum(deg, 1.0)), 0.0)
    dinv_ref[...] = dinv
    g = dinv * a7_ref[0]
    g_ref[0] = g[:, :HH]
    g_ref[1] = g[:, HH:]


def _pre(deg_parts, a):
    return pl.pallas_call(
        _pre_body,
        grid=(N // BD,),
        in_specs=[
            pl.BlockSpec((NC, BD, 16), lambda i: (0, i, 0)),
            pl.BlockSpec((1, BD, H), lambda i: (K - 1, i, 0)),
        ],
        out_specs=[
            pl.BlockSpec((BD, 1), lambda i: (i, 0)),
            pl.BlockSpec((NC, BD, HH), lambda i: (0, i, 0)),
        ],
        out_shape=[
            jax.ShapeDtypeStruct((N, 1), jnp.float32),
            jax.ShapeDtypeStruct((NC, N, HH), jnp.float32),
        ],
    )(deg_parts, a)


def _dense_body(a_ref, s_ref, dinv_ref, bk2_ref, bk_ref, g_ref):
    ssum = jnp.concatenate([s_ref[0], s_ref[1]], axis=-1)
    dinv = dinv_ref[...]
    bk = a_ref[0] - 2.0 * dinv * ssum - bk2_ref[...]
    bk_ref[...] = bk
    g = dinv * bk
    g_ref[0] = g[:, :HH]
    g_ref[1] = g[:, HH:]


def _dense_body_nob(a_ref, s_ref, dinv_ref, bk_ref, g_ref):
    ssum = jnp.concatenate([s_ref[0], s_ref[1]], axis=-1)
    dinv = dinv_ref[...]
    bk = a_ref[0] - 2.0 * dinv * ssum
    bk_ref[...] = bk
    g = dinv * bk
    g_ref[0] = g[:, :HH]
    g_ref[1] = g[:, HH:]


def _dense(k, a, s_parts, dinv, bk2):
    in_specs = [
        pl.BlockSpec((1, BD, H), lambda i, k=k: (k, i, 0)),
        pl.BlockSpec((NC, BD, HH), lambda i: (0, i, 0)),
        pl.BlockSpec((BD, 1), lambda i: (i, 0)),
    ]
    args = [a, s_parts, dinv]
    if bk2 is None:
        body = _dense_body_nob
    else:
        body = _dense_body
        in_specs.append(pl.BlockSpec((BD, H), lambda i: (i, 0)))
        args.append(bk2)
    return pl.pallas_call(
        body,
        grid=(N // BD,),
        in_specs=in_specs,
        out_specs=[
            pl.BlockSpec((BD, H), lambda i: (i, 0)),
            pl.BlockSpec((NC, BD, HH), lambda i: (0, i, 0)),
        ],
        out_shape=[
            jax.ShapeDtypeStruct((N, H), jnp.float32),
            jax.ShapeDtypeStruct((NC, N, HH), jnp.float32),
        ],
    )(*args)


def _final_body(a_ref, s_ref, dinv_ref, b2_ref, bias_ref, o_ref):
    ssum = jnp.concatenate([s_ref[0], s_ref[1]], axis=-1)
    o_ref[...] = jnp.maximum(
        a_ref[0] - dinv_ref[...] * ssum - b2_ref[...] + bias_ref[...], 0.0)


def _final(a, s_parts, dinv, b2, bias2d):
    return pl.pallas_call(
        _final_body,
        grid=(N // BD,),
        in_specs=[
            pl.BlockSpec((1, BD, H), lambda i: (0, i, 0)),
            pl.BlockSpec((NC, BD, HH), lambda i: (0, i, 0)),
            pl.BlockSpec((BD, 1), lambda i: (i, 0)),
            pl.BlockSpec((BD, H), lambda i: (i, 0)),
            pl.BlockSpec((1, H), lambda i: (0, 0)),
        ],
        out_specs=pl.BlockSpec((BD, H), lambda i: (i, 0)),
        out_shape=jax.ShapeDtypeStruct((N, H), jnp.float32),
    )(a, s_parts, dinv, b2, bias2d)


def kernel(x, edge_index, W, b):
    src = edge_index[0].astype(jnp.int32)
    dst = edge_index[1].astype(jnp.int32)
    pad = EP - E
    shape4 = (NC, NS, NCH, CH)
    src_s = jnp.concatenate([src, jnp.full((pad,), TRASH, jnp.int32)]).reshape(shape4)
    pad2 = EP2 - E
    shape3 = (NS, NCH2, CH)
    src_g = jnp.concatenate([src, jnp.zeros((pad2,), jnp.int32)]).reshape(shape3)
    dst_s = jnp.concatenate([dst, jnp.full((pad2,), TRASH, jnp.int32)]).reshape(shape3)

    bias2d = b.reshape(1, H)
    zeros_h = jnp.zeros((ACC, HH), jnp.float32)
    zeros_16 = jnp.zeros((ACC, 16), jnp.float32)
    ones_16 = jnp.ones((CH, 16), jnp.float32)

    deg_parts = _sc_deg(src_s, ones_16, zeros_16)
    a = _matmul(x, W)
    dinv, g = _pre(deg_parts, a)

    b_prev2 = None
    b_prev1 = a[K - 1]
    for k in range(K - 2, 0, -1):
        s_parts = _sc_prop(g, src_g, dst_s, zeros_h)
        bk, g = _dense(k, a, s_parts, dinv, b_prev2)
        b_prev2, b_prev1 = b_prev1, bk

    s_parts = _sc_prop(g, src_g, dst_s, zeros_h)
    return _final(a, s_parts, dinv, b_prev2, bias2d)
